# baseline (speedup 1.0000x reference)
import functools

import jax
import jax.numpy as jnp
from jax import lax
from jax.experimental import pallas as pl
from jax.experimental.pallas import tpu as pltpu
from jax.experimental.pallas import tpu_sc as plsc

N_TOK = 36864
C = 8192
D = 64
N_BLK = 2048


LANES = 128
CHUNK = 512
ROWT = 256


def _dist_argmin_body(x_ref, e_ref, idx_ref, esq_ref, et_ref):
    @pl.when(pl.program_id(0) == 0)
    def _():
        et = e_ref[...].T
        et_ref[...] = et
        esq_ref[...] = jnp.sum(et * et, axis=0, keepdims=True)

    x = x_ref[...]
    x_sq = jnp.sum(x * x, axis=-1, keepdims=True)
    xx = x + x
    lane = lax.broadcasted_iota(jnp.int32, (ROWT, LANES), 1).astype(jnp.float32)
    for r in range(N_BLK // ROWT):
        rs = slice(r * ROWT, (r + 1) * ROWT)
        xr = xx[rs]
        xs = x_sq[rs]
        best_v = jnp.full((ROWT, LANES), jnp.inf, jnp.float32)
        best_t = jnp.zeros((ROWT, LANES), jnp.float32)
        for k in range(C // CHUNK):
            cs = slice(k * CHUNK, (k + 1) * CHUNK)
            twocross = jnp.dot(xr, et_ref[:, cs],
                               preferred_element_type=jnp.float32)
            for j in range(CHUNK // LANES):
                js = slice(k * CHUNK + j * LANES, k * CHUNK + (j + 1) * LANES)
                dv = (xs + esq_ref[:, js]) - twocross[:, j * LANES:(j + 1) * LANES]
                t = jnp.float32(k * (CHUNK // LANES) + j)
                upd = dv < best_v
                best_v = jnp.where(upd, dv, best_v)
                best_t = jnp.where(upd, t, best_t)
        cfull = best_t * jnp.float32(LANES) + lane
        m = jnp.min(best_v, axis=-1, keepdims=True)
        idxf = jnp.min(jnp.where(best_v <= m, cfull, jnp.float32(C)),
                       axis=-1, keepdims=True)
        idx_ref[rs, :] = idxf.astype(jnp.int32)


def _dist_argmin(x2d, table):
    grid = (N_TOK // N_BLK,)
    return pl.pallas_call(
        _dist_argmin_body,
        grid=grid,
        in_specs=[
            pl.BlockSpec((N_BLK, D), lambda i: (i, 0)),
            pl.BlockSpec((C, D), lambda i: (0, 0)),
        ],
        out_specs=pl.BlockSpec((N_BLK, 1), lambda i: (i, 0)),
        out_shape=jax.ShapeDtypeStruct((N_TOK, 1), jnp.int32),
        scratch_shapes=[pltpu.VMEM((1, C), jnp.float32),
                        pltpu.VMEM((D, C), jnp.float32)],
    )(x2d, table)


@functools.cache
def _sc_gather_fn():
    info = plsc.get_sparse_core_info()
    nc = info.num_cores
    nw = nc * info.num_subcores
    b_per_w = N_TOK // nw

    @functools.partial(
        pl.kernel,
        out_type=jax.ShapeDtypeStruct((N_TOK, D), jnp.float32),
        mesh=plsc.VectorSubcoreMesh(core_axis_name="c", subcore_axis_name="s"),
        compiler_params=pltpu.CompilerParams(use_tc_tiling_on_sc=False),
        scratch_types=[
            pltpu.VMEM((b_per_w,), jnp.int32),
            pltpu.VMEM((b_per_w, D), jnp.float32),
            pltpu.SemaphoreType.DMA,
        ],
    )
    def _sc_gather(table_hbm, idx_hbm, out_hbm, idx_v, rows_v, sem):
        wid = lax.axis_index("s") * nc + lax.axis_index("c")
        base = wid * b_per_w
        pltpu.sync_copy(idx_hbm.at[pl.ds(base, b_per_w)], idx_v)
        pltpu.async_copy(table_hbm.at[idx_v], rows_v, sem).wait()
        pltpu.sync_copy(rows_v, out_hbm.at[pl.ds(base, b_per_w)])

    return _sc_gather


def kernel(x, embed):
    x2d = x[0].astype(jnp.float32)
    table = embed[0].astype(jnp.float32)
    idx = _dist_argmin(x2d, table)
    out = _sc_gather_fn()(table, idx.reshape(N_TOK))
    return out[None]


# speedup vs baseline: 1.0508x; 1.0508x over previous
import functools

import jax
import jax.numpy as jnp
from jax import lax
from jax.experimental import pallas as pl
from jax.experimental.pallas import tpu as pltpu
from jax.experimental.pallas import tpu_sc as plsc

N_TOK = 36864
C = 8192
D = 64
N_BLK = 2048


LANES = 128
CHUNK = 1024


def _dist_argmin_body(x_ref, et_ref, idx_ref, esq_ref):
    @pl.when(pl.program_id(0) == 0)
    def _():
        et = et_ref[...]
        esq_ref[...] = jnp.sum(et * et, axis=0, keepdims=True)

    x = x_ref[...]
    x_sq = jnp.sum(x * x, axis=-1, keepdims=True)
    xx = x + x
    best_v = jnp.full((N_BLK, LANES), jnp.inf, jnp.float32)
    best_t = jnp.zeros((N_BLK, LANES), jnp.float32)
    for k in range(C // CHUNK):
        cs = slice(k * CHUNK, (k + 1) * CHUNK)
        twocross = jnp.dot(xx, et_ref[:, cs],
                           preferred_element_type=jnp.float32)
        d = (x_sq + esq_ref[:, cs]) - twocross
        for j in range(CHUNK // LANES):
            dv = d[:, j * LANES:(j + 1) * LANES]
            t = jnp.float32(k * (CHUNK // LANES) + j)
            upd = dv < best_v
            best_v = jnp.where(upd, dv, best_v)
            best_t = jnp.where(upd, t, best_t)
    lane = lax.broadcasted_iota(jnp.int32, (N_BLK, LANES), 1).astype(jnp.float32)
    cfull = best_t * jnp.float32(LANES) + lane
    m = jnp.min(best_v, axis=-1, keepdims=True)
    idxf = jnp.min(jnp.where(best_v <= m, cfull, jnp.float32(C)),
                   axis=-1, keepdims=True)
    idx_ref[...] = idxf.astype(jnp.int32)


def _dist_argmin(x2d, embed_t):
    grid = (N_TOK // N_BLK,)
    return pl.pallas_call(
        _dist_argmin_body,
        grid=grid,
        in_specs=[
            pl.BlockSpec((N_BLK, D), lambda i: (i, 0)),
            pl.BlockSpec((D, C), lambda i: (0, 0)),
        ],
        out_specs=pl.BlockSpec((N_BLK, 1), lambda i: (i, 0)),
        out_shape=jax.ShapeDtypeStruct((N_TOK, 1), jnp.int32),
        scratch_shapes=[pltpu.VMEM((1, C), jnp.float32)],
    )(x2d, embed_t)


@functools.cache
def _sc_gather_fn():
    info = plsc.get_sparse_core_info()
    nc = info.num_cores
    nw = nc * info.num_subcores
    b_per_w = N_TOK // nw

    @functools.partial(
        pl.kernel,
        out_type=jax.ShapeDtypeStruct((N_TOK, D), jnp.float32),
        mesh=plsc.VectorSubcoreMesh(core_axis_name="c", subcore_axis_name="s"),
        compiler_params=pltpu.CompilerParams(use_tc_tiling_on_sc=False),
        scratch_types=[
            pltpu.VMEM((b_per_w,), jnp.int32),
            pltpu.VMEM((b_per_w, D), jnp.float32),
            pltpu.SemaphoreType.DMA,
        ],
    )
    def _sc_gather(table_hbm, idx_hbm, out_hbm, idx_v, rows_v, sem):
        wid = lax.axis_index("s") * nc + lax.axis_index("c")
        base = wid * b_per_w
        pltpu.sync_copy(idx_hbm.at[pl.ds(base, b_per_w)], idx_v)
        pltpu.async_copy(table_hbm.at[idx_v], rows_v, sem).wait()
        pltpu.sync_copy(rows_v, out_hbm.at[pl.ds(base, b_per_w)])

    return _sc_gather


def kernel(x, embed):
    x2d = x[0].astype(jnp.float32)
    table = embed[0].astype(jnp.float32)
    idx = _dist_argmin(x2d, table.T)
    out = _sc_gather_fn()(table, idx.reshape(N_TOK))
    return out[None]
